# Initial kernel scaffold; baseline (speedup 1.0000x reference)
import jax
import jax.numpy as jnp
from jax.experimental import pallas as pl


def kernel(x, indices_0, indices_1, weights):
    raise NotImplementedError("write your pallas kernel here")


# speedup vs baseline: 1.0059x; 1.0059x over previous
import functools

import jax
import jax.numpy as jnp
from jax import lax
from jax.experimental import pallas as pl
from jax.experimental.pallas import tpu as pltpu
from jax.experimental.pallas import tpu_sc as plsc

IN_DIM = 32768
OUT_DIM = 32768
BATCH = 512

NC = 2
NS = 16
NW = NC * NS
LANES = 16

P = OUT_DIM // NW
C = 64
NCHUNK = P // C
NVEC = BATCH // LANES

TBLK = 512


def _tr_body(x_ref, o_ref):
    o_ref[...] = x_ref[...].T


def _transpose_in(x):
    return pl.pallas_call(
        _tr_body,
        grid=(IN_DIM // TBLK,),
        in_specs=[pl.BlockSpec((BATCH, TBLK), lambda i: (0, i))],
        out_specs=pl.BlockSpec((TBLK, BATCH), lambda i: (i, 0)),
        out_shape=jax.ShapeDtypeStruct((IN_DIM, BATCH), jnp.float32),
    )(x)


def _transpose_out(yT):
    return pl.pallas_call(
        _tr_body,
        grid=(OUT_DIM // TBLK,),
        in_specs=[pl.BlockSpec((TBLK, BATCH), lambda i: (i, 0))],
        out_specs=pl.BlockSpec((BATCH, TBLK), lambda i: (0, i)),
        out_shape=jax.ShapeDtypeStruct((BATCH, OUT_DIM), jnp.float32),
    )(yT)


_mesh = plsc.VectorSubcoreMesh(
    core_axis_name="c", subcore_axis_name="s", num_cores=NC, num_subcores=NS
)


@functools.partial(
    pl.kernel,
    out_type=jax.ShapeDtypeStruct((OUT_DIM, BATCH), jnp.float32),
    mesh=_mesh,
    compiler_params=pltpu.CompilerParams(needs_layout_passes=False),
    scratch_types=[
        pltpu.VMEM((C,), jnp.int32),
        pltpu.VMEM((C,), jnp.int32),
        pltpu.VMEM((C, 16), jnp.float32),
        pltpu.VMEM((C,), jnp.float32),
        pltpu.VMEM((C,), jnp.float32),
        pltpu.VMEM((C,), jnp.float32),
        pltpu.VMEM((C,), jnp.float32),
        pltpu.VMEM((C, BATCH), jnp.float32),
        pltpu.VMEM((C, BATCH), jnp.float32),
        pltpu.VMEM((C, BATCH), jnp.float32),
        pltpu.SemaphoreType.DMA,
        pltpu.SemaphoreType.DMA,
    ],
)
def _sc_gather_mix(xT, idx0, idx1, w, out,
                   idx0_v, idx1_v, w_v, c0_v, ca_v, cb_v, cab_v,
                   a_v, b_v, o_v, sem_a, sem_b):
    wid = lax.axis_index("s") * NC + lax.axis_index("c")
    base0 = wid * P
    lane = lax.iota(jnp.int32, LANES)

    def chunk(ci, carry):
        base = base0 + ci * C
        pltpu.sync_copy(idx0.at[pl.ds(base, C)], idx0_v)
        pltpu.sync_copy(idx1.at[pl.ds(base, C)], idx1_v)
        pltpu.sync_copy(w.at[pl.ds(base, C)], w_v)
        acp = pltpu.make_async_copy(xT.at[idx0_v], a_v, sem_a)
        bcp = pltpu.make_async_copy(xT.at[idx1_v], b_v, sem_b)
        acp.start()
        bcp.start()

        for q in range(C // LANES):
            rows = lane + q * LANES
            e = []
            for g in range(16):
                cols = jnp.full((LANES,), g, jnp.int32)
                e.append(jnp.exp(plsc.load_gather(w_v, [rows, cols])))
            s = (((e[0] + e[1]) + (e[2] + e[3])) + ((e[4] + e[5]) + (e[6] + e[7]))) + (
                ((e[8] + e[9]) + (e[10] + e[11])) + ((e[12] + e[13]) + (e[14] + e[15]))
            )
            inv = 1.0 / s
            t89 = e[8] + e[9]
            c0 = (t89 + (e[10] + e[11])) + ((e[12] + e[13]) + (e[14] + e[15]))
            ca = ((e[2] + e[3]) + (e[6] + e[7])) - (t89 + (e[12] + e[13]))
            cb = ((e[4] + e[5]) + (e[6] + e[7])) - (t89 + (e[10] + e[11]))
            cab = ((e[1] - e[2]) + (e[8] - e[4])) + ((e[11] - e[7]) + (e[13] - e[14])) \
                + 2.0 * (e[9] - e[6])
            sl = pl.ds(q * LANES, LANES)
            c0_v[sl] = c0 * inv
            ca_v[sl] = ca * inv
            cb_v[sl] = cb * inv
            cab_v[sl] = cab * inv

        acp.wait()
        bcp.wait()

        def neuron(jj, _):
            jx = jnp.full((LANES,), jj, jnp.int32)
            c0b = plsc.load_gather(c0_v, [jx])
            cab_b = plsc.load_gather(cab_v, [jx])
            ca_b = plsc.load_gather(ca_v, [jx])
            cb_b = plsc.load_gather(cb_v, [jx])
            for v in range(NVEC):
                sl = pl.ds(v * LANES, LANES)
                a = a_v[jj, sl]
                b = b_v[jj, sl]
                o_v[jj, sl] = (c0b + a * ca_b) + b * (cb_b + a * cab_b)
            return 0

        lax.fori_loop(0, C, neuron, 0)
        pltpu.sync_copy(o_v, out.at[pl.ds(base, C)])
        return carry

    lax.fori_loop(0, NCHUNK, chunk, 0)


def kernel(x, indices_0, indices_1, weights):
    xT = _transpose_in(x)
    yT = _sc_gather_mix(xT, indices_0, indices_1, weights)
    return _transpose_out(yT)


# speedup vs baseline: 1.5922x; 1.5829x over previous
import functools

import jax
import jax.numpy as jnp
from jax import lax
from jax.experimental import pallas as pl
from jax.experimental.pallas import tpu as pltpu
from jax.experimental.pallas import tpu_sc as plsc

IN_DIM = 32768
OUT_DIM = 32768
BATCH = 512

NC = 2
NS = 16
NW = NC * NS
LANES = 16

P = OUT_DIM // NW
C = 32
NCHUNK = P // C
NVEC = BATCH // LANES

TBLK = 2048


def _tr_body(x_ref, o_ref):
    o_ref[...] = x_ref[...].T


def _transpose_in(x):
    return pl.pallas_call(
        _tr_body,
        grid=(IN_DIM // TBLK,),
        in_specs=[pl.BlockSpec((BATCH, TBLK), lambda i: (0, i))],
        out_specs=pl.BlockSpec((TBLK, BATCH), lambda i: (i, 0)),
        out_shape=jax.ShapeDtypeStruct((IN_DIM, BATCH), jnp.float32),
    )(x)


def _transpose_out(yT):
    return pl.pallas_call(
        _tr_body,
        grid=(OUT_DIM // TBLK,),
        in_specs=[pl.BlockSpec((TBLK, BATCH), lambda i: (i, 0))],
        out_specs=pl.BlockSpec((BATCH, TBLK), lambda i: (0, i)),
        out_shape=jax.ShapeDtypeStruct((BATCH, OUT_DIM), jnp.float32),
    )(yT)


_mesh = plsc.VectorSubcoreMesh(
    core_axis_name="c", subcore_axis_name="s", num_cores=NC, num_subcores=NS
)


@functools.partial(
    pl.kernel,
    out_type=jax.ShapeDtypeStruct((OUT_DIM, BATCH), jnp.float32),
    mesh=_mesh,
    compiler_params=pltpu.CompilerParams(needs_layout_passes=False),
    scratch_types=[
        pltpu.VMEM((P,), jnp.int32),
        pltpu.VMEM((P,), jnp.int32),
        pltpu.VMEM((P * 16,), jnp.float32),
        pltpu.VMEM((P,), jnp.float32),
        pltpu.VMEM((P,), jnp.float32),
        pltpu.VMEM((P,), jnp.float32),
        pltpu.VMEM((P,), jnp.float32),
        pltpu.VMEM((C, BATCH), jnp.float32),
        pltpu.VMEM((C, BATCH), jnp.float32),
        pltpu.VMEM((C, BATCH), jnp.float32),
        pltpu.VMEM((C, BATCH), jnp.float32),
        pltpu.VMEM((C, BATCH), jnp.float32),
        pltpu.VMEM((C, BATCH), jnp.float32),
        pltpu.SemaphoreType.DMA,
        pltpu.SemaphoreType.DMA,
        pltpu.SemaphoreType.DMA,
        pltpu.SemaphoreType.DMA,
        pltpu.SemaphoreType.DMA,
        pltpu.SemaphoreType.DMA,
    ],
)
def _sc_gather_mix(xT, idx0, idx1, w, out,
                   idx0_w, idx1_w, w_w, c0_v, ca_v, cb_v, cab_v,
                   a0, b0, a1, b1, o0, o1,
                   sem_a0, sem_b0, sem_a1, sem_b1, sem_o0, sem_o1):
    wid = lax.axis_index("s") * NC + lax.axis_index("c")
    base0 = wid * P
    lane = lax.iota(jnp.int32, LANES)

    def gather_start(ci, a_buf, b_buf, sem_a, sem_b):
        sl = pl.ds(ci * C, C)
        pltpu.make_async_copy(xT.at[idx0_w.at[sl]], a_buf, sem_a).start()
        pltpu.make_async_copy(xT.at[idx1_w.at[sl]], b_buf, sem_b).start()

    def gather_wait(a_buf, b_buf, sem_a, sem_b):
        pltpu.make_async_copy(xT.at[idx0_w.at[pl.ds(0, C)]], a_buf, sem_a).wait()
        pltpu.make_async_copy(xT.at[idx1_w.at[pl.ds(0, C)]], b_buf, sem_b).wait()

    def out_copy(ci, o_buf, sem_o):
        return pltpu.make_async_copy(
            o_buf, out.at[pl.ds(base0 + ci * C, C)], sem_o)

    pltpu.sync_copy(idx0.at[pl.ds(base0, P)], idx0_w)
    pltpu.sync_copy(idx1.at[pl.ds(base0, P)], idx1_w)
    gather_start(0, a0, b0, sem_a0, sem_b0)
    gather_start(1, a1, b1, sem_a1, sem_b1)
    pltpu.sync_copy(w.at[pl.ds(base0 * 16, P * 16)], w_w)

    lane16 = lane * 16

    def coef_group(q, _):
        e = []
        for g in range(16):
            e.append(jnp.exp(plsc.load_gather(w_w, [lane16 + (q * 256 + g)])))
        s = (((e[0] + e[1]) + (e[2] + e[3])) + ((e[4] + e[5]) + (e[6] + e[7]))) + (
            ((e[8] + e[9]) + (e[10] + e[11])) + ((e[12] + e[13]) + (e[14] + e[15]))
        )
        inv = 1.0 / s
        t89 = e[8] + e[9]
        c0 = (t89 + (e[10] + e[11])) + ((e[12] + e[13]) + (e[14] + e[15]))
        ca = ((e[2] + e[3]) + (e[6] + e[7])) - (t89 + (e[12] + e[13]))
        cb = ((e[4] + e[5]) + (e[6] + e[7])) - (t89 + (e[10] + e[11]))
        cab = ((e[1] - e[2]) + (e[8] - e[4])) + ((e[11] - e[7]) + (e[13] - e[14])) \
            + 2.0 * (e[9] - e[6])
        sl = pl.ds(q * LANES, LANES)
        c0_v[sl] = c0 * inv
        ca_v[sl] = ca * inv
        cb_v[sl] = cb * inv
        cab_v[sl] = cab * inv
        return 0

    lax.fori_loop(0, P // LANES, coef_group, 0)

    def mix(ci, a_buf, b_buf, o_buf):
        def neuron(jj, _):
            jx = jnp.full((LANES,), ci * C + jj, jnp.int32)
            c0b = plsc.load_gather(c0_v, [jx])
            cab_b = plsc.load_gather(cab_v, [jx])
            ca_b = plsc.load_gather(ca_v, [jx])
            cb_b = plsc.load_gather(cb_v, [jx])
            for v in range(NVEC):
                sl = pl.ds(v * LANES, LANES)
                a = a_buf[jj, sl]
                b = b_buf[jj, sl]
                o_buf[jj, sl] = (c0b + a * ca_b) + b * (cb_b + a * cab_b)
            return 0

        lax.fori_loop(0, C, neuron, 0)

    def pair(k, _):
        ci = 2 * k
        gather_wait(a0, b0, sem_a0, sem_b0)

        @pl.when(k > 0)
        def _():
            out_copy(ci, o0, sem_o0).wait()

        mix(ci, a0, b0, o0)
        out_copy(ci, o0, sem_o0).start()

        @pl.when(k < NCHUNK // 2 - 1)
        def _():
            gather_start(ci + 2, a0, b0, sem_a0, sem_b0)

        cj = 2 * k + 1
        gather_wait(a1, b1, sem_a1, sem_b1)

        @pl.when(k > 0)
        def _():
            out_copy(cj, o1, sem_o1).wait()

        mix(cj, a1, b1, o1)
        out_copy(cj, o1, sem_o1).start()

        @pl.when(k < NCHUNK // 2 - 1)
        def _():
            gather_start(cj + 2, a1, b1, sem_a1, sem_b1)

        return 0

    lax.fori_loop(0, NCHUNK // 2, pair, 0)
    out_copy(NCHUNK - 2, o0, sem_o0).wait()
    out_copy(NCHUNK - 1, o1, sem_o1).wait()


def kernel(x, indices_0, indices_1, weights):
    xT = _transpose_in(x)
    yT = _sc_gather_mix(xT, indices_0, indices_1, weights.reshape(-1))
    return _transpose_out(yT)


# speedup vs baseline: 1.6125x; 1.0127x over previous
import functools

import jax
import jax.numpy as jnp
from jax import lax
from jax.experimental import pallas as pl
from jax.experimental.pallas import tpu as pltpu
from jax.experimental.pallas import tpu_sc as plsc

IN_DIM = 32768
OUT_DIM = 32768
BATCH = 512

NC = 2
NS = 16
NW = NC * NS
LANES = 16

P = OUT_DIM // NW
C = 32
NCHUNK = P // C
NVEC = BATCH // LANES

TBLK = 4096


def _tr_body(x_ref, o_ref):
    o_ref[...] = x_ref[...].T


def _transpose_in(x):
    return pl.pallas_call(
        _tr_body,
        grid=(IN_DIM // TBLK,),
        in_specs=[pl.BlockSpec((BATCH, TBLK), lambda i: (0, i))],
        out_specs=pl.BlockSpec((TBLK, BATCH), lambda i: (i, 0)),
        out_shape=jax.ShapeDtypeStruct((IN_DIM, BATCH), jnp.float32),
    )(x)


def _transpose_out(yT):
    return pl.pallas_call(
        _tr_body,
        grid=(OUT_DIM // TBLK,),
        in_specs=[pl.BlockSpec((TBLK, BATCH), lambda i: (i, 0))],
        out_specs=pl.BlockSpec((BATCH, TBLK), lambda i: (0, i)),
        out_shape=jax.ShapeDtypeStruct((BATCH, OUT_DIM), jnp.float32),
    )(yT)


_mesh = plsc.VectorSubcoreMesh(
    core_axis_name="c", subcore_axis_name="s", num_cores=NC, num_subcores=NS
)


@functools.partial(
    pl.kernel,
    out_type=jax.ShapeDtypeStruct((OUT_DIM, BATCH), jnp.float32),
    mesh=_mesh,
    compiler_params=pltpu.CompilerParams(needs_layout_passes=False),
    scratch_types=[
        pltpu.VMEM((P,), jnp.int32),
        pltpu.VMEM((P,), jnp.int32),
        pltpu.VMEM((P * 16,), jnp.float32),
        pltpu.VMEM((P,), jnp.float32),
        pltpu.VMEM((P,), jnp.float32),
        pltpu.VMEM((P,), jnp.float32),
        pltpu.VMEM((P,), jnp.float32),
        pltpu.VMEM((C, BATCH), jnp.float32),
        pltpu.VMEM((C, BATCH), jnp.float32),
        pltpu.VMEM((C, BATCH), jnp.float32),
        pltpu.VMEM((C, BATCH), jnp.float32),
        pltpu.VMEM((C, BATCH), jnp.float32),
        pltpu.VMEM((C, BATCH), jnp.float32),
        pltpu.SemaphoreType.DMA,
        pltpu.SemaphoreType.DMA,
        pltpu.SemaphoreType.DMA,
        pltpu.SemaphoreType.DMA,
        pltpu.SemaphoreType.DMA,
        pltpu.SemaphoreType.DMA,
    ],
)
def _sc_gather_mix(xT, idx0, idx1, w, out,
                   idx0_w, idx1_w, w_w, c0_v, ca_v, cb_v, cab_v,
                   a0, b0, a1, b1, o0, o1,
                   sem_a0, sem_b0, sem_a1, sem_b1, sem_o0, sem_o1):
    wid = lax.axis_index("s") * NC + lax.axis_index("c")
    base0 = wid * P
    lane = lax.iota(jnp.int32, LANES)

    def gather_start(ci, a_buf, b_buf, sem_a, sem_b):
        sl = pl.ds(ci * C, C)
        pltpu.make_async_copy(xT.at[idx0_w.at[sl]], a_buf, sem_a).start()
        pltpu.make_async_copy(xT.at[idx1_w.at[sl]], b_buf, sem_b).start()

    def gather_wait(a_buf, b_buf, sem_a, sem_b):
        pltpu.make_async_copy(xT.at[idx0_w.at[pl.ds(0, C)]], a_buf, sem_a).wait()
        pltpu.make_async_copy(xT.at[idx1_w.at[pl.ds(0, C)]], b_buf, sem_b).wait()

    def out_copy(ci, o_buf, sem_o):
        return pltpu.make_async_copy(
            o_buf, out.at[pl.ds(base0 + ci * C, C)], sem_o)

    pltpu.sync_copy(idx0.at[pl.ds(base0, P)], idx0_w)
    pltpu.sync_copy(idx1.at[pl.ds(base0, P)], idx1_w)
    gather_start(0, a0, b0, sem_a0, sem_b0)
    gather_start(1, a1, b1, sem_a1, sem_b1)
    pltpu.sync_copy(w.at[pl.ds(base0 * 16, P * 16)], w_w)

    lane16 = lane * 16

    def coef_group(q, _):
        e = []
        for g in range(16):
            e.append(jnp.exp(plsc.load_gather(w_w, [lane16 + (q * 256 + g)])))
        s = (((e[0] + e[1]) + (e[2] + e[3])) + ((e[4] + e[5]) + (e[6] + e[7]))) + (
            ((e[8] + e[9]) + (e[10] + e[11])) + ((e[12] + e[13]) + (e[14] + e[15]))
        )
        inv = 1.0 / s
        t89 = e[8] + e[9]
        c0 = (t89 + (e[10] + e[11])) + ((e[12] + e[13]) + (e[14] + e[15]))
        ca = ((e[2] + e[3]) + (e[6] + e[7])) - (t89 + (e[12] + e[13]))
        cb = ((e[4] + e[5]) + (e[6] + e[7])) - (t89 + (e[10] + e[11]))
        cab = ((e[1] - e[2]) + (e[8] - e[4])) + ((e[11] - e[7]) + (e[13] - e[14])) \
            + 2.0 * (e[9] - e[6])
        sl = pl.ds(q * LANES, LANES)
        c0_v[sl] = c0 * inv
        ca_v[sl] = ca * inv
        cb_v[sl] = cb * inv
        cab_v[sl] = cab * inv
        return 0

    lax.fori_loop(0, P // LANES, coef_group, 0)

    def mix(ci, a_buf, b_buf, o_buf):
        def neuron(jj, _):
            jx = jnp.full((LANES,), ci * C + jj, jnp.int32)
            c0b = plsc.load_gather(c0_v, [jx])
            cab_b = plsc.load_gather(cab_v, [jx])
            ca_b = plsc.load_gather(ca_v, [jx])
            cb_b = plsc.load_gather(cb_v, [jx])
            for v in range(NVEC):
                sl = pl.ds(v * LANES, LANES)
                a = a_buf[jj, sl]
                b = b_buf[jj, sl]
                o_buf[jj, sl] = (c0b + a * ca_b) + b * (cb_b + a * cab_b)
            return 0

        lax.fori_loop(0, C, neuron, 0)

    def pair(k, _):
        ci = 2 * k
        gather_wait(a0, b0, sem_a0, sem_b0)

        @pl.when(k > 0)
        def _():
            out_copy(ci, o0, sem_o0).wait()

        mix(ci, a0, b0, o0)
        out_copy(ci, o0, sem_o0).start()

        @pl.when(k < NCHUNK // 2 - 1)
        def _():
            gather_start(ci + 2, a0, b0, sem_a0, sem_b0)

        cj = 2 * k + 1
        gather_wait(a1, b1, sem_a1, sem_b1)

        @pl.when(k > 0)
        def _():
            out_copy(cj, o1, sem_o1).wait()

        mix(cj, a1, b1, o1)
        out_copy(cj, o1, sem_o1).start()

        @pl.when(k < NCHUNK // 2 - 1)
        def _():
            gather_start(cj + 2, a1, b1, sem_a1, sem_b1)

        return 0

    lax.fori_loop(0, NCHUNK // 2, pair, 0)
    out_copy(NCHUNK - 2, o0, sem_o0).wait()
    out_copy(NCHUNK - 1, o1, sem_o1).wait()


def kernel(x, indices_0, indices_1, weights):
    xT = _transpose_in(x)
    yT = _sc_gather_mix(xT, indices_0, indices_1, weights.reshape(-1))
    return _transpose_out(yT)
